# Initial kernel scaffold; baseline (speedup 1.0000x reference)
import jax
import jax.numpy as jnp
from jax.experimental import pallas as pl


def kernel(x, edge_index, edge_weight):
    raise NotImplementedError("write your pallas kernel here")


# speedup vs baseline: 2.1373x; 2.1373x over previous
import functools

import jax
import jax.numpy as jnp
from jax import lax
from jax.experimental import pallas as pl
from jax.experimental.pallas import tpu as pltpu
from jax.experimental.pallas import tpu_sc as plsc

N_NODES = 10000
N_EDGES = 320000
D_FEAT = 128
K_HOPS = 2

NUM_CORES = 2
NUM_SUBCORES = 16
NUM_WORKERS = NUM_CORES * NUM_SUBCORES
F_PER = D_FEAT // NUM_WORKERS

EDGE_CHUNK = 4000
N_CHUNKS = N_EDGES // EDGE_CHUNK
VECS_PER_CHUNK = EDGE_CHUNK // 16


def _make_sgconv():
  mesh = plsc.VectorSubcoreMesh(core_axis_name="c", subcore_axis_name="s")

  @functools.partial(
      pl.kernel,
      mesh=mesh,
      out_type=jax.ShapeDtypeStruct((D_FEAT, N_NODES), jnp.float32),
      compiler_params=pltpu.CompilerParams(needs_layout_passes=False),
      scratch_types=(
          [pltpu.VMEM((N_NODES,), jnp.float32) for _ in range(F_PER)]
          + [pltpu.VMEM((N_NODES,), jnp.float32) for _ in range(F_PER)]
          + [
              pltpu.VMEM((EDGE_CHUNK,), jnp.int32),
              pltpu.VMEM((EDGE_CHUNK,), jnp.int32),
              pltpu.VMEM((EDGE_CHUNK,), jnp.float32),
          ]
      ),
  )
  def sgconv(xt_hbm, src_hbm, dst_hbm, w_hbm, out_hbm, *scratch):
    a_bufs = scratch[:F_PER]
    b_bufs = scratch[F_PER:2 * F_PER]
    src_v, dst_v, w_v = scratch[2 * F_PER:]

    wid = lax.axis_index("c") * NUM_SUBCORES + lax.axis_index("s")
    f0 = wid * F_PER

    for f in range(F_PER):
      pltpu.sync_copy(xt_hbm.at[f0 + f], a_bufs[f])

    def zero_bufs(bufs):
      zeros = jnp.zeros((16,), jnp.float32)
      def body(i, _):
        for buf in bufs:
          buf[pl.ds(i * 16, 16)] = zeros
        return 0
      lax.fori_loop(0, N_NODES // 16, body, 0)

    def edge_pass(from_bufs, to_bufs):
      def chunk_body(ci, _):
        base = ci * EDGE_CHUNK
        pltpu.sync_copy(src_hbm.at[pl.ds(base, EDGE_CHUNK)], src_v)
        pltpu.sync_copy(dst_hbm.at[pl.ds(base, EDGE_CHUNK)], dst_v)
        pltpu.sync_copy(w_hbm.at[pl.ds(base, EDGE_CHUNK)], w_v)

        def vec_body(i, _):
          s = src_v[pl.ds(i * 16, 16)]
          d = dst_v[pl.ds(i * 16, 16)]
          w = w_v[pl.ds(i * 16, 16)]
          for f in range(F_PER):
            vals = plsc.load_gather(from_bufs[f], [s])
            plsc.addupdate_scatter(to_bufs[f], [d], vals * w)
          return 0

        lax.fori_loop(0, VECS_PER_CHUNK, vec_body, 0)
        return 0

      lax.fori_loop(0, N_CHUNKS, chunk_body, 0)

    zero_bufs(b_bufs)
    edge_pass(a_bufs, b_bufs)
    zero_bufs(a_bufs)
    edge_pass(b_bufs, a_bufs)
    for f in range(F_PER):
      pltpu.sync_copy(a_bufs[f], out_hbm.at[f0 + f])

  return sgconv


_sgconv = _make_sgconv()


@jax.jit
def kernel(x, edge_index, edge_weight):
  xt = x.T
  dst = edge_index[0]
  src = edge_index[1]
  out_t = _sgconv(xt, src, dst, edge_weight)
  return out_t.T


# speedup vs baseline: 4.1167x; 1.9261x over previous
import functools

import jax
import jax.numpy as jnp
from jax import lax
from jax.experimental import pallas as pl
from jax.experimental.pallas import tpu as pltpu
from jax.experimental.pallas import tpu_sc as plsc

N_NODES = 10000
N_EDGES = 320000
D_FEAT = 128
K_HOPS = 2

NUM_CORES = 2
NUM_SUBCORES = 16
NUM_WORKERS = NUM_CORES * NUM_SUBCORES
F_PER = D_FEAT // NUM_WORKERS

EDGE_CHUNK = 4000
N_CHUNKS = N_EDGES // EDGE_CHUNK
VECS_PER_CHUNK = EDGE_CHUNK // 16


def _make_sgconv():
  mesh = plsc.VectorSubcoreMesh(core_axis_name="c", subcore_axis_name="s")

  @functools.partial(
      pl.kernel,
      mesh=mesh,
      out_type=jax.ShapeDtypeStruct((D_FEAT, N_NODES), jnp.float32),
      compiler_params=pltpu.CompilerParams(needs_layout_passes=False),
      scratch_types=(
          [pltpu.VMEM((N_NODES,), jnp.float32) for _ in range(F_PER)]
          + [pltpu.VMEM((N_NODES,), jnp.float32) for _ in range(F_PER)]
          + [
              pltpu.VMEM((EDGE_CHUNK,), jnp.int32),
              pltpu.VMEM((EDGE_CHUNK,), jnp.int32),
              pltpu.VMEM((EDGE_CHUNK,), jnp.float32),
          ]
      ),
  )
  def sgconv(xt_hbm, src_hbm, dst_hbm, w_hbm, out_hbm, *scratch):
    a_bufs = scratch[:F_PER]
    b_bufs = scratch[F_PER:2 * F_PER]
    src_v, dst_v, w_v = scratch[2 * F_PER:]

    wid = lax.axis_index("c") * NUM_SUBCORES + lax.axis_index("s")
    f0 = wid * F_PER

    for f in range(F_PER):
      pltpu.sync_copy(xt_hbm.at[f0 + f], a_bufs[f])

    def zero_bufs(bufs):
      zeros = jnp.zeros((16,), jnp.float32)
      def body(i, _):
        for buf in bufs:
          buf[pl.ds(i * 16, 16)] = zeros
        return 0
      lax.fori_loop(0, N_NODES // 16, body, 0)

    def edge_pass(from_bufs, to_bufs):
      def chunk_body(ci, _):
        base = ci * EDGE_CHUNK
        pltpu.sync_copy(src_hbm.at[pl.ds(base, EDGE_CHUNK)], src_v)
        pltpu.sync_copy(dst_hbm.at[pl.ds(base, EDGE_CHUNK)], dst_v)
        pltpu.sync_copy(w_hbm.at[pl.ds(base, EDGE_CHUNK)], w_v)

        @plsc.parallel_loop(0, VECS_PER_CHUNK, unroll=4)
        def vec_body(i):
          s = src_v[pl.ds(i * 16, 16)]
          d = dst_v[pl.ds(i * 16, 16)]
          w = w_v[pl.ds(i * 16, 16)]
          for f in range(F_PER):
            vals = plsc.load_gather(from_bufs[f], [s])
            plsc.addupdate_scatter(to_bufs[f], [d], vals * w)

        return 0

      lax.fori_loop(0, N_CHUNKS, chunk_body, 0)

    zero_bufs(b_bufs)
    edge_pass(a_bufs, b_bufs)
    zero_bufs(a_bufs)
    edge_pass(b_bufs, a_bufs)
    for f in range(F_PER):
      pltpu.sync_copy(a_bufs[f], out_hbm.at[f0 + f])

  return sgconv


_sgconv = _make_sgconv()


@jax.jit
def kernel(x, edge_index, edge_weight):
  xt = x.T
  dst = edge_index[0]
  src = edge_index[1]
  out_t = _sgconv(xt, src, dst, edge_weight)
  return out_t.T


# speedup vs baseline: 7.1474x; 1.7362x over previous
import functools

import jax
import jax.numpy as jnp
from jax import lax
from jax.experimental import pallas as pl
from jax.experimental.pallas import tpu as pltpu
from jax.experimental.pallas import tpu_sc as plsc

N_NODES = 10000
N_EDGES = 320000
D_FEAT = 128
K_HOPS = 2

NUM_CORES = 2
NUM_SUBCORES = 16
NUM_WORKERS = NUM_CORES * NUM_SUBCORES
F_PER = D_FEAT // NUM_WORKERS

EDGE_CHUNK = 4000
N_CHUNKS = N_EDGES // EDGE_CHUNK
VECS_PER_CHUNK = EDGE_CHUNK // 16


def _make_sgconv():
  mesh = plsc.VectorSubcoreMesh(core_axis_name="c", subcore_axis_name="s")

  @functools.partial(
      pl.kernel,
      mesh=mesh,
      out_type=jax.ShapeDtypeStruct((D_FEAT, N_NODES), jnp.float32),
      compiler_params=pltpu.CompilerParams(needs_layout_passes=False),
      scratch_types=(
          [pltpu.VMEM((N_NODES,), jnp.float32) for _ in range(F_PER)]
          + [pltpu.VMEM((N_NODES,), jnp.float32) for _ in range(F_PER)]
          + [pltpu.VMEM((EDGE_CHUNK,), jnp.int32) for _ in range(2)]
          + [pltpu.VMEM((EDGE_CHUNK,), jnp.int32) for _ in range(2)]
          + [pltpu.VMEM((EDGE_CHUNK,), jnp.float32) for _ in range(2)]
          + [pltpu.SemaphoreType.DMA, pltpu.SemaphoreType.DMA]
      ),
  )
  def sgconv(xt_hbm, src_hbm, dst_hbm, w_hbm, out_hbm, *scratch):
    a_bufs = scratch[:F_PER]
    b_bufs = scratch[F_PER:2 * F_PER]
    n = 2 * F_PER
    src_bufs = scratch[n:n + 2]
    dst_bufs = scratch[n + 2:n + 4]
    w_bufs = scratch[n + 4:n + 6]
    sems = scratch[n + 6:n + 8]

    wid = lax.axis_index("c") * NUM_SUBCORES + lax.axis_index("s")
    f0 = wid * F_PER

    for f in range(F_PER):
      pltpu.sync_copy(xt_hbm.at[f0 + f], a_bufs[f])

    def zero_bufs(bufs):
      zeros = jnp.zeros((16,), jnp.float32)
      def body(i, _):
        for buf in bufs:
          buf[pl.ds(i * 16, 16)] = zeros
        return 0
      lax.fori_loop(0, N_NODES // 16, body, 0)

    def issue_fetch(ci, b):
      base = ci * EDGE_CHUNK
      pltpu.async_copy(src_hbm.at[pl.ds(base, EDGE_CHUNK)], src_bufs[b],
                       sems[b])
      pltpu.async_copy(dst_hbm.at[pl.ds(base, EDGE_CHUNK)], dst_bufs[b],
                       sems[b])
      pltpu.async_copy(w_hbm.at[pl.ds(base, EDGE_CHUNK)], w_bufs[b], sems[b])

    def drain_fetch(b):
      pltpu.make_async_copy(src_hbm.at[pl.ds(0, EDGE_CHUNK)], src_bufs[b],
                            sems[b]).wait()
      pltpu.make_async_copy(dst_hbm.at[pl.ds(0, EDGE_CHUNK)], dst_bufs[b],
                            sems[b]).wait()
      pltpu.make_async_copy(w_hbm.at[pl.ds(0, EDGE_CHUNK)], w_bufs[b],
                            sems[b]).wait()

    def edge_pass(from_bufs, to_bufs):
      for b in range(2):
        issue_fetch(b, b)

      def chunk_pair_body(g, _):
        for b in range(2):
          ci = g * 2 + b
          drain_fetch(b)
          src_v, dst_v, w_v = src_bufs[b], dst_bufs[b], w_bufs[b]

          @plsc.parallel_loop(0, VECS_PER_CHUNK, unroll=8)
          def vec_body(i):
            s = src_v[pl.ds(i * 16, 16)]
            d = dst_v[pl.ds(i * 16, 16)]
            w = w_v[pl.ds(i * 16, 16)]
            for f in range(F_PER):
              vals = plsc.load_gather(from_bufs[f], [s])
              plsc.addupdate_scatter(to_bufs[f], [d], vals * w)

          @pl.when(ci + 2 < N_CHUNKS)
          def _():
            issue_fetch(ci + 2, b)

        return 0

      lax.fori_loop(0, N_CHUNKS // 2, chunk_pair_body, 0)

    zero_bufs(b_bufs)
    edge_pass(a_bufs, b_bufs)
    zero_bufs(a_bufs)
    edge_pass(b_bufs, a_bufs)
    for f in range(F_PER):
      pltpu.sync_copy(a_bufs[f], out_hbm.at[f0 + f])

  return sgconv


_sgconv = _make_sgconv()


@jax.jit
def kernel(x, edge_index, edge_weight):
  xt = x.T
  dst = edge_index[0]
  src = edge_index[1]
  out_t = _sgconv(xt, src, dst, edge_weight)
  return out_t.T
